# baseline (speedup 1.0000x reference)
import functools

import jax
import jax.numpy as jnp
from jax import lax
from jax.experimental import pallas as pl
from jax.experimental.pallas import tpu as pltpu
from jax.experimental.pallas import tpu_sc as plsc

B = 1024
L = 200
H = 128
HALF = H // 2
POS_ROWS = 513
SEG_ROWS = 3
FUSED_ROWS = SEG_ROWS * POS_ROWS
N = B * L
NSUB = 16
PER_S = N // NSUB
C = 128
NCHUNK = PER_S // C
NGRP = C // 16


def _fused_tc_body(tok_ref, pos_ref, seg_ref, out_ref):
    tp = tok_ref[...] + pos_ref[...]
    out_ref[...] = tp[None, :, :] + seg_ref[...][:, None, :]


def _sc_body(x_hbm, xseg_hbm, f3a_hbm, f3b_hbm, out_hbm,
             f3, xc0, xc1, sc0, sc1, st0, st1,
             si0, si1, so0, so1):
    cid = lax.axis_index("c")
    sid = lax.axis_index("s")
    rowbase = sid * PER_S
    colbase = cid * HALF

    @pl.when(cid == 0)
    def _():
        pltpu.sync_copy(f3a_hbm, f3)

    @pl.when(cid == 1)
    def _():
        pltpu.sync_copy(f3b_hbm, f3)

    xcs = (xc0, xc1)
    scs = (sc0, sc1)
    stages = (st0, st1)
    sis = (si0, si1)
    sos = (so0, so1)
    lane = jnp.arange(16, dtype=jnp.int32)

    def issue_idx(i, b):
        pltpu.async_copy(x_hbm.at[sid, i], xcs[b], sis[b])
        pltpu.async_copy(xseg_hbm.at[sid, i], scs[b], sis[b])

    def wait_idx(b):
        pltpu.make_async_copy(x_hbm.at[sid, 0], xcs[b], sis[b]).wait()
        pltpu.make_async_copy(xseg_hbm.at[sid, 0], scs[b], sis[b]).wait()

    def out_dst(i):
        return out_hbm.at[pl.ds(rowbase + i * C, C), pl.ds(colbase, HALF)]

    def wait_out(b):
        pltpu.make_async_copy(stages[b], out_dst(0), sos[b]).wait()

    def expand_chunk(b):
        xc, sc, st = xcs[b], scs[b], stages[b]

        @plsc.parallel_loop(0, NGRP, unroll=2)
        def grp(g):
            pv = xc[pl.ds(g * 16, 16)]
            sv = sc[pl.ds(g * 16, 16)]
            spvec = sv * POS_ROWS + pv
            rvec = g * 16 + lane
            for c in range(HALF):
                cvec = jnp.full((16,), c, jnp.int32)
                tv = plsc.load_gather(f3, [spvec, cvec])
                plsc.store_scatter(st, [rvec, cvec], tv)

    issue_idx(0, 0)

    def step(k, carry):
        for b in range(2):
            i = 2 * k + b
            wait_idx(b)

            @pl.when(i + 1 < NCHUNK)
            def _():
                issue_idx(i + 1, 1 - b)

            @pl.when(k > 0)
            def _():
                wait_out(b)

            expand_chunk(b)
            pltpu.async_copy(stages[b], out_dst(i), sos[b])
        return carry

    lax.fori_loop(0, NCHUNK // 2, step, 0)
    wait_out(0)
    wait_out(1)


@jax.jit
def _run(x3d, xseg3d, tok513, pos_table, seg_table):
    fused = pl.pallas_call(
        _fused_tc_body,
        out_shape=jax.ShapeDtypeStruct((SEG_ROWS, POS_ROWS, H), jnp.float32),
    )(tok513, pos_table, seg_table)
    ff = fused.reshape(FUSED_ROWS, H)
    f3a = ff[:, :HALF]
    f3b = ff[:, HALF:]

    mesh = plsc.VectorSubcoreMesh(core_axis_name="c", subcore_axis_name="s")
    call = pl.kernel(
        _sc_body,
        out_type=jax.ShapeDtypeStruct((N, H), jnp.float32),
        mesh=mesh,
        compiler_params=pltpu.CompilerParams(use_tc_tiling_on_sc=False, needs_layout_passes=False),
        scratch_types=[
            pltpu.VMEM((FUSED_ROWS, HALF), jnp.float32),
            pltpu.VMEM((C,), jnp.int32),
            pltpu.VMEM((C,), jnp.int32),
            pltpu.VMEM((C,), jnp.int32),
            pltpu.VMEM((C,), jnp.int32),
            pltpu.VMEM((C, HALF), jnp.float32),
            pltpu.VMEM((C, HALF), jnp.float32),
            pltpu.SemaphoreType.DMA,
            pltpu.SemaphoreType.DMA,
            pltpu.SemaphoreType.DMA,
            pltpu.SemaphoreType.DMA,
        ],
    )
    return call(x3d, xseg3d, f3a, f3b)


def kernel(x, x_seg, token_table, pos_table, seg_table):
    x3d = x.reshape(NSUB, NCHUNK, C)
    xseg3d = x_seg.reshape(NSUB, NCHUNK, C)
    out = _run(x3d, xseg3d, token_table[:POS_ROWS], pos_table, seg_table)
    return out.reshape(B, L, H)


# speedup vs baseline: 3.8817x; 3.8817x over previous
import functools

import jax
import jax.numpy as jnp
from jax import lax
from jax.experimental import pallas as pl
from jax.experimental.pallas import tpu as pltpu
from jax.experimental.pallas import tpu_sc as plsc

B = 1024
L = 200
H = 128
POS_ROWS = 513
SEG_ROWS = 3
FUSED_ROWS = SEG_ROWS * POS_ROWS
N = B * L
NW = 32
PER_W = N // NW
C = 80
NCHUNK = PER_W // C
NGRP = C // 16
NCOL = H // 16
KMAIN = 26


def _fused_tc_body(tok_ref, pos_ref, seg_ref, out_ref):
    tp = tok_ref[...] + pos_ref[...]
    out_ref[...] = tp[None, :, :] + seg_ref[...][:, None, :]


def _sc_body(x_hbm, xseg_hbm, fused_hbm, tokpos_hbm, seg_hbm, out_hbm,
             xi, si, ci, tp, sg, b0, b1, b2,
             sg1, sg2, so0, so1, so2):
    wid = lax.axis_index("s") * 2 + lax.axis_index("c")
    base = wid * PER_W
    pltpu.sync_copy(x_hbm.at[wid], xi)
    pltpu.sync_copy(xseg_hbm.at[wid], si)
    pltpu.sync_copy(tokpos_hbm, tp)
    pltpu.sync_copy(seg_hbm, sg)

    segv = [[sg[s, pl.ds(j * 16, 16)] for j in range(NCOL)]
            for s in range(SEG_ROWS)]

    def mkidx(r, carry):
        for j in range(NGRP):
            sl = (r, pl.ds(j * 16, 16))
            ci[sl] = si[sl] * POS_ROWS + xi[sl]
        return carry

    lax.fori_loop(0, NCHUNK, mkidx, 0)

    bufs = (b0, b1, b2)
    sgs = (None, sg1, sg2)
    sos = (so0, so1, so2)

    def issue(i, b):
        pltpu.async_copy(fused_hbm.at[ci.at[i]], bufs[b], sgs[b])

    def wait_gather(b):
        pltpu.make_async_copy(fused_hbm.at[ci.at[0]], bufs[b], sgs[b]).wait()

    def start_out(i, b):
        pltpu.async_copy(bufs[b], out_hbm.at[pl.ds(base + i * C, C)], sos[b])

    def wait_out(b):
        pltpu.make_async_copy(
            bufs[b], out_hbm.at[pl.ds(base, C)], sos[b]).wait()

    def expand_chunk(i):
        st = bufs[0]

        @plsc.parallel_loop(0, NGRP, unroll=2)
        def grp(g):
            pvec = xi[i, pl.ds(g * 16, 16)]
            svec = si[i, pl.ds(g * 16, 16)]
            for l in range(16):
                p = pvec[l]
                s = svec[l]
                m1 = s == 1
                m2 = s == 2
                for j in range(NCOL):
                    tv = tp[p, pl.ds(j * 16, 16)]
                    sv = jnp.where(m2, segv[2][j],
                                   jnp.where(m1, segv[1][j], segv[0][j]))
                    st[g * 16 + l, pl.ds(j * 16, 16)] = tv + sv

    issue(1, 1)
    issue(2, 2)

    def step(k, carry):
        i0 = 3 * k

        @pl.when(k > 0)
        def _():
            wait_out(0)

        expand_chunk(i0)
        start_out(i0, 0)

        @pl.when(k > 0)
        def _():
            wait_out(2)
            issue(i0 + 2, 2)

        wait_gather(1)
        start_out(i0 + 1, 1)

        wait_gather(2)
        start_out(i0 + 2, 2)
        wait_out(1)
        issue(i0 + 4, 1)
        return carry

    lax.fori_loop(0, KMAIN, step, 0)

    wait_out(0)
    expand_chunk(NCHUNK - 2)
    start_out(NCHUNK - 2, 0)
    wait_gather(1)
    start_out(NCHUNK - 1, 1)
    wait_out(0)
    wait_out(1)
    wait_out(2)


@jax.jit
def _run(x3d, xseg3d, tok513, pos_table, seg_table):
    fused = pl.pallas_call(
        _fused_tc_body,
        out_shape=jax.ShapeDtypeStruct((SEG_ROWS, POS_ROWS, H), jnp.float32),
    )(tok513, pos_table, seg_table)
    ff = fused.reshape(FUSED_ROWS, H)
    tokpos = ff[:POS_ROWS]

    mesh = plsc.VectorSubcoreMesh(core_axis_name="c", subcore_axis_name="s")
    call = pl.kernel(
        _sc_body,
        out_type=jax.ShapeDtypeStruct((N, H), jnp.float32),
        mesh=mesh,
        scratch_types=[
            pltpu.VMEM((NCHUNK, C), jnp.int32),
            pltpu.VMEM((NCHUNK, C), jnp.int32),
            pltpu.VMEM((NCHUNK, C), jnp.int32),
            pltpu.VMEM((POS_ROWS, H), jnp.float32),
            pltpu.VMEM((SEG_ROWS, H), jnp.float32),
            pltpu.VMEM((C, H), jnp.float32),
            pltpu.VMEM((C, H), jnp.float32),
            pltpu.VMEM((C, H), jnp.float32),
            pltpu.SemaphoreType.DMA,
            pltpu.SemaphoreType.DMA,
            pltpu.SemaphoreType.DMA,
            pltpu.SemaphoreType.DMA,
            pltpu.SemaphoreType.DMA,
        ],
    )
    return call(x3d, xseg3d, ff, tokpos, seg_table)


def kernel(x, x_seg, token_table, pos_table, seg_table):
    x3d = x.reshape(NW, NCHUNK, C)
    xseg3d = x_seg.reshape(NW, NCHUNK, C)
    out = _run(x3d, xseg3d, token_table[:POS_ROWS], pos_table, seg_table)
    return out.reshape(B, L, H)


# speedup vs baseline: 5.7310x; 1.4764x over previous
import functools

import jax
import jax.numpy as jnp
from jax import lax
from jax.experimental import pallas as pl
from jax.experimental.pallas import tpu as pltpu
from jax.experimental.pallas import tpu_sc as plsc

B = 1024
L = 200
H = 128
POS_ROWS = 513
SEG_ROWS = 3
N = B * L
NW = 32
PER_W = N // NW
C = 128
NCHUNK = PER_W // C
NBUF = 6
NCOL = H // 16


def _fused_tc_body(tok_ref, pos_ref, seg_ref, out_ref):
    tp = tok_ref[...] + pos_ref[...]
    out_ref[...] = tp[None, :, :] + seg_ref[...][:, None, :]


def _sc_body(x_hbm, xseg_hbm, fused_hbm, out_hbm,
             xi, si, b0, b1, b2, b3, b4, b5,
             sg0, sg1, sg2, sg3, sg4, sg5,
             so0, so1, so2, so3, so4, so5):
    wid = lax.axis_index("s") * 2 + lax.axis_index("c")
    base = wid * PER_W
    pltpu.sync_copy(x_hbm.at[wid], xi)
    pltpu.sync_copy(xseg_hbm.at[wid], si)

    def mkidx(r, carry):
        for j in range(NCOL):
            sl = (r, pl.ds(j * 16, 16))
            si[sl] = si[sl] * POS_ROWS + xi[sl]
        return carry

    lax.fori_loop(0, NCHUNK, mkidx, 0)

    bufs = (b0, b1, b2, b3, b4, b5)
    sgs = (sg0, sg1, sg2, sg3, sg4, sg5)
    sos = (so0, so1, so2, so3, so4, so5)

    def issue(i, b):
        pltpu.async_copy(fused_hbm.at[si.at[i]], bufs[b], sgs[b])

    def wait_gather(b):
        pltpu.make_async_copy(fused_hbm.at[si.at[0]], bufs[b], sgs[b]).wait()

    def wait_out(b):
        pltpu.make_async_copy(
            bufs[b], out_hbm.at[pl.ds(base, C)], sos[b]).wait()

    for b in range(NBUF - 1):
        issue(b, b)

    def step(k, carry):
        for b in range(NBUF):
            i = NBUF * k + b
            wait_gather(b)
            pltpu.async_copy(
                bufs[b], out_hbm.at[pl.ds(base + i * C, C)], sos[b])
            nxt = (b + NBUF - 1) % NBUF

            @pl.when(NBUF * k + b + NBUF - 1 < NCHUNK)
            def _():
                @pl.when(k + b > 0)
                def _():
                    wait_out(nxt)

                issue(i + NBUF - 1, nxt)
        return carry

    lax.fori_loop(0, NCHUNK // NBUF, step, 0)
    for i in range(NBUF * (NCHUNK // NBUF), NCHUNK):
        b = i % NBUF
        wait_gather(b)
        pltpu.async_copy(
            bufs[b], out_hbm.at[pl.ds(base + i * C, C)], sos[b])
    for i in range(NCHUNK - NBUF, NCHUNK):
        wait_out(i % NBUF)


@jax.jit
def _run(x3d, xseg3d, tok513, pos_table, seg_table):
    fused = pl.pallas_call(
        _fused_tc_body,
        out_shape=jax.ShapeDtypeStruct((SEG_ROWS, POS_ROWS, H), jnp.float32),
    )(tok513, pos_table, seg_table)
    fused = fused.reshape(SEG_ROWS * POS_ROWS, H)

    mesh = plsc.VectorSubcoreMesh(core_axis_name="c", subcore_axis_name="s")
    call = pl.kernel(
        _sc_body,
        out_type=jax.ShapeDtypeStruct((N, H), jnp.float32),
        mesh=mesh,
        scratch_types=[
            pltpu.VMEM((NCHUNK, C), jnp.int32),
            pltpu.VMEM((NCHUNK, C), jnp.int32),
            pltpu.VMEM((C, H), jnp.float32),
            pltpu.VMEM((C, H), jnp.float32),
            pltpu.VMEM((C, H), jnp.float32),
            pltpu.VMEM((C, H), jnp.float32),
            pltpu.VMEM((C, H), jnp.float32),
            pltpu.VMEM((C, H), jnp.float32),
            pltpu.SemaphoreType.DMA,
            pltpu.SemaphoreType.DMA,
            pltpu.SemaphoreType.DMA,
            pltpu.SemaphoreType.DMA,
            pltpu.SemaphoreType.DMA,
            pltpu.SemaphoreType.DMA,
            pltpu.SemaphoreType.DMA,
            pltpu.SemaphoreType.DMA,
            pltpu.SemaphoreType.DMA,
            pltpu.SemaphoreType.DMA,
            pltpu.SemaphoreType.DMA,
            pltpu.SemaphoreType.DMA,
        ],
    )
    return call(x3d, xseg3d, fused)


def kernel(x, x_seg, token_table, pos_table, seg_table):
    x3d = x.reshape(NW, NCHUNK, C)
    xseg3d = x_seg.reshape(NW, NCHUNK, C)
    out = _run(x3d, xseg3d, token_table[:POS_ROWS], pos_table, seg_table)
    return out.reshape(B, L, H)
